# baseline (speedup 1.0000x reference)
import functools

import jax
import jax.numpy as jnp
from jax import lax
from jax.experimental import pallas as pl
from jax.experimental.pallas import tpu as pltpu
from jax.experimental.pallas import tpu_sc as plsc

N = 10000
E = 320000
D = 128

NC = 2
NS = 16
NW = NC * NS

GC = 40
SC_ = GC
CHUNKS = [(0, 3760, 94), (120320, 3760, 94), (240640, 2480, 62)]
K = len(CHUNKS)

NP = 10240
NPC = NP // NS
ZR = 64

_sc_mesh = plsc.VectorSubcoreMesh(core_axis_name="c", subcore_axis_name="s")


NSLOT = 4


def _make_gather(chunk):
  ebase, EPW, GNCHUNK = CHUNKS[chunk]

  @functools.partial(
      pl.kernel,
      out_type=jax.ShapeDtypeStruct((EPW * NW, D), jnp.float32),
      mesh=_sc_mesh,
      scratch_types=[
          pltpu.VMEM((EPW,), jnp.int32),
          pltpu.VMEM((EPW,), jnp.int32),
          [pltpu.VMEM((GC, D), jnp.float32)] * NSLOT,
          [pltpu.VMEM((GC, D), jnp.float32)] * NSLOT,
          [pltpu.VMEM((GC, D), jnp.float32)] * NSLOT,
          [pltpu.VMEM((GC, D), jnp.float32)] * NSLOT,
          [pltpu.SemaphoreType.DMA] * NSLOT,
          [pltpu.SemaphoreType.DMA] * NSLOT,
          pltpu.SemaphoreType.DMA,
      ],
  )
  def _sc_gather(x_hbm, s_hbm, r_hbm, ea_hbm, out_hbm,
                 idx_s, idx_r, ea_v, xs_v, xr_v, o_v, sem_in, sem_out,
                 sem_idx):
    wid = lax.axis_index("s") * NC + lax.axis_index("c")
    base = ebase + wid * EPW
    obase = wid * EPW

    cp_s = pltpu.async_copy(s_hbm.at[pl.ds(base, EPW)], idx_s, sem_idx)
    cp_r = pltpu.async_copy(r_hbm.at[pl.ds(base, EPW)], idx_r, sem_idx)
    cp_s.wait()
    cp_r.wait()

    def issue_in(s, k):
        ioff = k * GC
        pltpu.async_copy(ea_hbm.at[pl.ds(base + k * GC, GC)], ea_v[s],
                         sem_in[s])
        pltpu.async_copy(x_hbm.at[idx_s.at[pl.ds(ioff, GC)]], xs_v[s],
                         sem_in[s])
        pltpu.async_copy(x_hbm.at[idx_r.at[pl.ds(ioff, GC)]], xr_v[s],
                         sem_in[s])

    def wait_in(s):
        pltpu.make_async_copy(ea_hbm.at[pl.ds(0, GC)], ea_v[s],
                              sem_in[s]).wait()
        pltpu.make_async_copy(ea_hbm.at[pl.ds(0, GC)], xs_v[s],
                              sem_in[s]).wait()
        pltpu.make_async_copy(ea_hbm.at[pl.ds(0, GC)], xr_v[s],
                              sem_in[s]).wait()

    def wait_out(s):
        pltpu.make_async_copy(o_v[s], out_hbm.at[pl.ds(0, GC)],
                              sem_out[s]).wait()

    def add_and_store(s, k):
        def row_body(i, _):
            for j in range(D // 16):
                sl = pl.ds(j * 16, 16)
                o_v[s][i, sl] = (ea_v[s][i, sl] + xs_v[s][i, sl]
                                 + xr_v[s][i, sl])
            return 0

        lax.fori_loop(0, GC, row_body, 0)
        pltpu.async_copy(o_v[s], out_hbm.at[pl.ds(obase + k * GC, GC)],
                         sem_out[s])

    for s in range(NSLOT):
        issue_in(s, s)

    def quad_body(j, _):
        for s in range(NSLOT):
            k = NSLOT * j + s
            wait_in(s)

            @pl.when(j >= 1)
            def _():
                wait_out(s)

            add_and_store(s, k)

            @pl.when(k + NSLOT < GNCHUNK)
            def _():
                issue_in(s, k + NSLOT)

        return 0

    lax.fori_loop(0, GNCHUNK // NSLOT, quad_body, 0)
    for t in range(GNCHUNK % NSLOT):
        wait_in(t)
        wait_out(t)
        add_and_store(t, GNCHUNK - GNCHUNK % NSLOT + t)
    for s in range(NSLOT):
        wait_out(s)

  return _sc_gather


_gathers = [_make_gather(c) for c in range(K)]


NSLOT_S = 4


def _make_scatter(chunk):
  ebase, SEPW, SNCHUNK = CHUNKS[chunk]

  @functools.partial(
      pl.kernel,
      out_type=jax.ShapeDtypeStruct((NC, NP, D), jnp.float32),
      mesh=_sc_mesh,
      scratch_types=[
          pltpu.VMEM_SHARED((NP, D), jnp.float32),
          [pltpu.VMEM((SC_,), jnp.int32)] * NSLOT_S,
          [pltpu.VMEM((SC_, D), jnp.float32)] * NSLOT_S,
          pltpu.VMEM((ZR, D), jnp.float32),
          [pltpu.SemaphoreType.DMA] * NSLOT_S,
      ],
  )
  def _sc_scatter(ne_hbm, r_hbm, out_hbm, agg_sh, idx_v, rows_v, zbuf,
                  sem_ld):
    cid = lax.axis_index("c")
    sid = lax.axis_index("s")
    wid = sid * NC + cid
    base = wid * SEPW
    rbase = ebase + base

    def issue_ld(s, k):
        pltpu.async_copy(r_hbm.at[pl.ds(rbase + k * SC_, SC_)], idx_v[s],
                         sem_ld[s])
        pltpu.async_copy(ne_hbm.at[pl.ds(base + k * SC_, SC_)], rows_v[s],
                         sem_ld[s])

    def wait_ld(s):
        pltpu.make_async_copy(r_hbm.at[pl.ds(0, SC_)], idx_v[s],
                              sem_ld[s]).wait()
        pltpu.make_async_copy(ne_hbm.at[pl.ds(0, SC_)], rows_v[s],
                              sem_ld[s]).wait()

    def scat(s, k):
        pltpu.sync_copy(rows_v[s], agg_sh.at[idx_v[s]], add=True)

    for s in range(NSLOT_S):
        issue_ld(s, s)

    zeros = jnp.zeros((16,), jnp.float32)

    def zrow(i, _):
        for j in range(D // 16):
            zbuf[i, pl.ds(j * 16, 16)] = zeros
        return 0

    lax.fori_loop(0, ZR, zrow, 0)
    for t in range(NPC // ZR):
        pltpu.sync_copy(zbuf, agg_sh.at[pl.ds(sid * NPC + t * ZR, ZR)])
    plsc.subcore_barrier()

    def ring_body(j, _):
        for s in range(NSLOT_S):
            k = NSLOT_S * j + s
            wait_ld(s)
            scat(s, k)

            @pl.when(k + NSLOT_S < SNCHUNK)
            def _():
                issue_ld(s, k + NSLOT_S)

        return 0

    lax.fori_loop(0, SNCHUNK // NSLOT_S, ring_body, 0)
    for t in range(SNCHUNK % NSLOT_S):
        wait_ld(t)
        scat(t, SNCHUNK - SNCHUNK % NSLOT_S + t)
    plsc.subcore_barrier()

    pltpu.sync_copy(agg_sh.at[pl.ds(sid * NPC, NPC)],
                    out_hbm.at[cid].at[pl.ds(sid * NPC, NPC)])

  return _sc_scatter


_scatters = [_make_scatter(c) for c in range(K)]


_BE = 2560
_BN = 1000


def _edge_mlp_body(buf_ref, e_ref, w1_ref, b1_ref, w2_ref, b2_ref,
                   o_ref, cp_ref):
    del buf_ref
    h = jnp.dot(e_ref[...], w1_ref[...], preferred_element_type=jnp.float32)
    h = jnp.maximum(h + b1_ref[...], 0.0)
    ne = (jnp.dot(h, w2_ref[...], preferred_element_type=jnp.float32)
          + b2_ref[...])
    o_ref[...] = ne
    cp_ref[...] = ne


def _node_mlp_body(x_ref, *rest):
    parts = rest[:2 * K]
    w1_ref, b1_ref, w2_ref, b2_ref, o_ref = rest[2 * K:]
    n = x_ref[...]
    for p in parts:
        n = n + p[0]
    h = jnp.dot(n, w1_ref[...], preferred_element_type=jnp.float32)
    h = jnp.maximum(h + b1_ref[...], 0.0)
    o_ref[...] = (
        jnp.dot(h, w2_ref[...], preferred_element_type=jnp.float32)
        + b2_ref[...]
    )


def _full(shape):
    return pl.BlockSpec(shape, lambda i: (0,) * len(shape))


def _edge_mlp_chunk(buf, e_in, We1, be1, We2, be2, chunk):
    ebase, epw, _ = CHUNKS[chunk]
    ne_c = epw * NW
    base = ebase // _BE
    return pl.pallas_call(
        _edge_mlp_body,
        grid=(ne_c // _BE,),
        in_specs=[
            pl.BlockSpec(memory_space=pl.ANY),
            pl.BlockSpec((_BE, D), lambda i: (i, 0)),
            _full((D, D)), _full((1, D)), _full((D, D)), _full((1, D)),
        ],
        out_specs=[
            pl.BlockSpec((_BE, D), lambda i: (base + i, 0)),
            pl.BlockSpec((_BE, D), lambda i: (i, 0)),
        ],
        out_shape=[
            jax.ShapeDtypeStruct((E, D), jnp.float32),
            jax.ShapeDtypeStruct((ne_c, D), jnp.float32),
        ],
        input_output_aliases={0: 0},
    )(buf, e_in, We1, be1.reshape(1, D), We2, be2.reshape(1, D))


def _edge_mlp_first(e_in, We1, be1, We2, be2):
    ne_c = CHUNKS[0][1] * NW

    def body(e_ref, w1_ref, b1_ref, w2_ref, b2_ref, o_ref, cp_ref):
        _edge_mlp_body(None, e_ref, w1_ref, b1_ref, w2_ref, b2_ref,
                       o_ref, cp_ref)

    return pl.pallas_call(
        body,
        grid=(ne_c // _BE,),
        in_specs=[
            pl.BlockSpec((_BE, D), lambda i: (i, 0)),
            _full((D, D)), _full((1, D)), _full((D, D)), _full((1, D)),
        ],
        out_specs=[
            pl.BlockSpec((_BE, D), lambda i: (i, 0)),
            pl.BlockSpec((_BE, D), lambda i: (i, 0)),
        ],
        out_shape=[
            jax.ShapeDtypeStruct((E, D), jnp.float32),
            jax.ShapeDtypeStruct((ne_c, D), jnp.float32),
        ],
    )(e_in, We1, be1.reshape(1, D), We2, be2.reshape(1, D))


def _node_mlp(x, parts, Wn1, bn1, Wn2, bn2):
    pspec = lambda c: pl.BlockSpec((1, _BN, D), lambda i, c=c: (c, i, 0))
    return pl.pallas_call(
        _node_mlp_body,
        grid=(N // _BN,),
        in_specs=(
            [pl.BlockSpec((_BN, D), lambda i: (i, 0))]
            + [pspec(c) for _ in range(K) for c in range(NC)]
            + [_full((D, D)), _full((1, D)), _full((D, D)), _full((1, D))]
        ),
        out_specs=pl.BlockSpec((_BN, D), lambda i: (i, 0)),
        out_shape=jax.ShapeDtypeStruct((N, D), jnp.float32),
    )(x, *[p for p in parts for _ in range(NC)],
      Wn1, bn1.reshape(1, D), Wn2, bn2.reshape(1, D))


def kernel(x, edge_index, edge_attr, We1, be1, We2, be2, Wn1, bn1, Wn2, bn2):
    senders = edge_index[0]
    receivers = edge_index[1]

    e_chunks = [_gathers[c](x, senders, receivers, edge_attr)
                for c in range(K)]

    new_edge, cp0 = _edge_mlp_first(e_chunks[0], We1, be1, We2, be2)
    parts = [_scatters[0](cp0, receivers)]
    for c in range(1, K):
        new_edge, cpc = _edge_mlp_chunk(new_edge, e_chunks[c], We1, be1,
                                        We2, be2, c)
        parts.append(_scatters[c](cpc, receivers))

    new_node = _node_mlp(x, parts, Wn1, bn1, Wn2, bn2)
    return new_node, new_edge


# speedup vs baseline: 1.0328x; 1.0328x over previous
import functools

import jax
import jax.numpy as jnp
from jax import lax
from jax.experimental import pallas as pl
from jax.experimental.pallas import tpu as pltpu
from jax.experimental.pallas import tpu_sc as plsc

N = 10000
E = 320000
D = 128

NC = 2
NS = 16
NW = NC * NS

GC = 40
SC_ = GC
CHUNKS = [(0, 3760, 94), (120320, 3760, 94), (240640, 2480, 62)]
K = len(CHUNKS)

NP = 10240
NPC = NP // NS
ZR = 64

_sc_mesh = plsc.VectorSubcoreMesh(core_axis_name="c", subcore_axis_name="s")


NSLOT = 4


def _make_gather(chunk):
  ebase, EPW, GNCHUNK = CHUNKS[chunk]

  @functools.partial(
      pl.kernel,
      out_type=jax.ShapeDtypeStruct((EPW * NW, D), jnp.float32),
      mesh=_sc_mesh,
      scratch_types=[
          pltpu.VMEM((EPW,), jnp.int32),
          pltpu.VMEM((EPW,), jnp.int32),
          [pltpu.VMEM((GC, D), jnp.float32)] * NSLOT,
          [pltpu.VMEM((GC, D), jnp.float32)] * NSLOT,
          [pltpu.VMEM((GC, D), jnp.float32)] * NSLOT,
          [pltpu.VMEM((GC, D), jnp.float32)] * NSLOT,
          [pltpu.SemaphoreType.DMA] * NSLOT,
          [pltpu.SemaphoreType.DMA] * NSLOT,
          pltpu.SemaphoreType.DMA,
      ],
  )
  def _sc_gather(x_hbm, s_hbm, r_hbm, ea_hbm, out_hbm,
                 idx_s, idx_r, ea_v, xs_v, xr_v, o_v, sem_in, sem_out,
                 sem_idx):
    wid = lax.axis_index("s") * NC + lax.axis_index("c")
    base = ebase + wid * EPW
    obase = wid * EPW

    cp_s = pltpu.async_copy(s_hbm.at[pl.ds(base, EPW)], idx_s, sem_idx)
    cp_r = pltpu.async_copy(r_hbm.at[pl.ds(base, EPW)], idx_r, sem_idx)
    cp_s.wait()
    cp_r.wait()

    def issue_in(s, k):
        ioff = k * GC
        pltpu.async_copy(ea_hbm.at[pl.ds(base + k * GC, GC)], ea_v[s],
                         sem_in[s])
        pltpu.async_copy(x_hbm.at[idx_s.at[pl.ds(ioff, GC)]], xs_v[s],
                         sem_in[s])
        pltpu.async_copy(x_hbm.at[idx_r.at[pl.ds(ioff, GC)]], xr_v[s],
                         sem_in[s])

    def wait_in(s):
        pltpu.make_async_copy(ea_hbm.at[pl.ds(0, GC)], ea_v[s],
                              sem_in[s]).wait()
        pltpu.make_async_copy(ea_hbm.at[pl.ds(0, GC)], xs_v[s],
                              sem_in[s]).wait()
        pltpu.make_async_copy(ea_hbm.at[pl.ds(0, GC)], xr_v[s],
                              sem_in[s]).wait()

    def wait_out(s):
        pltpu.make_async_copy(o_v[s], out_hbm.at[pl.ds(0, GC)],
                              sem_out[s]).wait()

    def add_and_store(s, k):
        def row_body(i, _):
            for j in range(D // 16):
                sl = pl.ds(j * 16, 16)
                o_v[s][i, sl] = (ea_v[s][i, sl] + xs_v[s][i, sl]
                                 + xr_v[s][i, sl])
            return 0

        lax.fori_loop(0, GC, row_body, 0)
        pltpu.async_copy(o_v[s], out_hbm.at[pl.ds(obase + k * GC, GC)],
                         sem_out[s])

    for s in range(NSLOT):
        issue_in(s, s)

    def quad_body(j, _):
        for s in range(NSLOT):
            k = NSLOT * j + s
            wait_in(s)

            @pl.when(j >= 1)
            def _():
                wait_out(s)

            add_and_store(s, k)

            @pl.when(k + NSLOT < GNCHUNK)
            def _():
                issue_in(s, k + NSLOT)

        return 0

    lax.fori_loop(0, GNCHUNK // NSLOT, quad_body, 0)
    for t in range(GNCHUNK % NSLOT):
        wait_in(t)
        wait_out(t)
        add_and_store(t, GNCHUNK - GNCHUNK % NSLOT + t)
    for s in range(NSLOT):
        wait_out(s)

  return _sc_gather


_gathers = [_make_gather(c) for c in range(K)]


NSLOT_S = 6


def _make_scatter(chunk):
  ebase, SEPW, SNCHUNK = CHUNKS[chunk]

  @functools.partial(
      pl.kernel,
      out_type=jax.ShapeDtypeStruct((NC, NP, D), jnp.float32),
      mesh=_sc_mesh,
      scratch_types=[
          pltpu.VMEM_SHARED((NP, D), jnp.float32),
          [pltpu.VMEM((SC_,), jnp.int32)] * NSLOT_S,
          [pltpu.VMEM((SC_, D), jnp.float32)] * NSLOT_S,
          pltpu.VMEM((ZR, D), jnp.float32),
          [pltpu.SemaphoreType.DMA] * NSLOT_S,
      ],
  )
  def _sc_scatter(ne_hbm, r_hbm, out_hbm, agg_sh, idx_v, rows_v, zbuf,
                  sem_ld):
    cid = lax.axis_index("c")
    sid = lax.axis_index("s")
    wid = sid * NC + cid
    base = wid * SEPW
    rbase = ebase + base

    def issue_ld(s, k):
        pltpu.async_copy(r_hbm.at[pl.ds(rbase + k * SC_, SC_)], idx_v[s],
                         sem_ld[s])
        pltpu.async_copy(ne_hbm.at[pl.ds(base + k * SC_, SC_)], rows_v[s],
                         sem_ld[s])

    def wait_ld(s):
        pltpu.make_async_copy(r_hbm.at[pl.ds(0, SC_)], idx_v[s],
                              sem_ld[s]).wait()
        pltpu.make_async_copy(ne_hbm.at[pl.ds(0, SC_)], rows_v[s],
                              sem_ld[s]).wait()

    def scat(s, k):
        pltpu.sync_copy(rows_v[s], agg_sh.at[idx_v[s]], add=True)

    for s in range(NSLOT_S):
        issue_ld(s, s)

    zeros = jnp.zeros((16,), jnp.float32)

    def zrow(i, _):
        for j in range(D // 16):
            zbuf[i, pl.ds(j * 16, 16)] = zeros
        return 0

    lax.fori_loop(0, ZR, zrow, 0)
    for t in range(NPC // ZR):
        pltpu.sync_copy(zbuf, agg_sh.at[pl.ds(sid * NPC + t * ZR, ZR)])
    plsc.subcore_barrier()

    def ring_body(j, _):
        for s in range(NSLOT_S):
            k = NSLOT_S * j + s
            wait_ld(s)
            scat(s, k)

            @pl.when(k + NSLOT_S < SNCHUNK)
            def _():
                issue_ld(s, k + NSLOT_S)

        return 0

    lax.fori_loop(0, SNCHUNK // NSLOT_S, ring_body, 0)
    for t in range(SNCHUNK % NSLOT_S):
        wait_ld(t)
        scat(t, SNCHUNK - SNCHUNK % NSLOT_S + t)
    plsc.subcore_barrier()

    pltpu.sync_copy(agg_sh.at[pl.ds(sid * NPC, NPC)],
                    out_hbm.at[cid].at[pl.ds(sid * NPC, NPC)])

  return _sc_scatter


_scatters = [_make_scatter(c) for c in range(K)]


_BE = 1280
_BN = 1000


def _edge_mlp_body(buf_ref, e_ref, w1_ref, b1_ref, w2_ref, b2_ref,
                   o_ref, cp_ref):
    del buf_ref
    h = jnp.dot(e_ref[...], w1_ref[...], preferred_element_type=jnp.float32)
    h = jnp.maximum(h + b1_ref[...], 0.0)
    ne = (jnp.dot(h, w2_ref[...], preferred_element_type=jnp.float32)
          + b2_ref[...])
    o_ref[...] = ne
    cp_ref[...] = ne


def _node_mlp_body(x_ref, *rest):
    parts = rest[:2 * K]
    w1_ref, b1_ref, w2_ref, b2_ref, o_ref = rest[2 * K:]
    n = x_ref[...]
    for p in parts:
        n = n + p[0]
    h = jnp.dot(n, w1_ref[...], preferred_element_type=jnp.float32)
    h = jnp.maximum(h + b1_ref[...], 0.0)
    o_ref[...] = (
        jnp.dot(h, w2_ref[...], preferred_element_type=jnp.float32)
        + b2_ref[...]
    )


def _full(shape):
    return pl.BlockSpec(shape, lambda i: (0,) * len(shape))


def _edge_mlp_chunk(buf, e_in, We1, be1, We2, be2, chunk):
    ebase, epw, _ = CHUNKS[chunk]
    ne_c = epw * NW
    base = ebase // _BE
    return pl.pallas_call(
        _edge_mlp_body,
        grid=(ne_c // _BE,),
        in_specs=[
            pl.BlockSpec(memory_space=pl.ANY),
            pl.BlockSpec((_BE, D), lambda i: (i, 0)),
            _full((D, D)), _full((1, D)), _full((D, D)), _full((1, D)),
        ],
        out_specs=[
            pl.BlockSpec((_BE, D), lambda i: (base + i, 0)),
            pl.BlockSpec((_BE, D), lambda i: (i, 0)),
        ],
        out_shape=[
            jax.ShapeDtypeStruct((E, D), jnp.float32),
            jax.ShapeDtypeStruct((ne_c, D), jnp.float32),
        ],
        input_output_aliases={0: 0},
    )(buf, e_in, We1, be1.reshape(1, D), We2, be2.reshape(1, D))


def _edge_mlp_first(e_in, We1, be1, We2, be2):
    ne_c = CHUNKS[0][1] * NW

    def body(e_ref, w1_ref, b1_ref, w2_ref, b2_ref, o_ref, cp_ref):
        _edge_mlp_body(None, e_ref, w1_ref, b1_ref, w2_ref, b2_ref,
                       o_ref, cp_ref)

    return pl.pallas_call(
        body,
        grid=(ne_c // _BE,),
        in_specs=[
            pl.BlockSpec((_BE, D), lambda i: (i, 0)),
            _full((D, D)), _full((1, D)), _full((D, D)), _full((1, D)),
        ],
        out_specs=[
            pl.BlockSpec((_BE, D), lambda i: (i, 0)),
            pl.BlockSpec((_BE, D), lambda i: (i, 0)),
        ],
        out_shape=[
            jax.ShapeDtypeStruct((E, D), jnp.float32),
            jax.ShapeDtypeStruct((ne_c, D), jnp.float32),
        ],
    )(e_in, We1, be1.reshape(1, D), We2, be2.reshape(1, D))


def _node_mlp(x, parts, Wn1, bn1, Wn2, bn2):
    pspec = lambda c: pl.BlockSpec((1, _BN, D), lambda i, c=c: (c, i, 0))
    return pl.pallas_call(
        _node_mlp_body,
        grid=(N // _BN,),
        in_specs=(
            [pl.BlockSpec((_BN, D), lambda i: (i, 0))]
            + [pspec(c) for _ in range(K) for c in range(NC)]
            + [_full((D, D)), _full((1, D)), _full((D, D)), _full((1, D))]
        ),
        out_specs=pl.BlockSpec((_BN, D), lambda i: (i, 0)),
        out_shape=jax.ShapeDtypeStruct((N, D), jnp.float32),
    )(x, *[p for p in parts for _ in range(NC)],
      Wn1, bn1.reshape(1, D), Wn2, bn2.reshape(1, D))


def kernel(x, edge_index, edge_attr, We1, be1, We2, be2, Wn1, bn1, Wn2, bn2):
    senders = edge_index[0]
    receivers = edge_index[1]

    e_chunks = [_gathers[c](x, senders, receivers, edge_attr)
                for c in range(K)]

    new_edge, cp0 = _edge_mlp_first(e_chunks[0], We1, be1, We2, be2)
    parts = [_scatters[0](cp0, receivers)]
    for c in range(1, K):
        new_edge, cpc = _edge_mlp_chunk(new_edge, e_chunks[c], We1, be1,
                                        We2, be2, c)
        parts.append(_scatters[c](cpc, receivers))

    new_node = _node_mlp(x, parts, Wn1, bn1, Wn2, bn2)
    return new_node, new_edge
